# baseline (device time: 103805 ns/iter reference)
import jax
import jax.numpy as jnp
from jax import lax
from jax.experimental import pallas as pl
from jax.experimental.pallas import tpu as pltpu


def kernel(x, w_mat):
    k_glob, kc = x.shape
    n_dev = k_glob // kc
    m_per = kc
    _, n = w_mat.shape

    def body(x_ref, w_ref, out_ref, xg_ref, amax_ref,
             xs_sem, xr_sem, as_sem, ar_sem):
        j = pl.program_id(0)
        me = lax.axis_index("i")

        def x_block(d):
            return x_ref.at[pl.ds(d * m_per, m_per), :]

        @pl.when(j == 0)
        def _():
            xg_ref[pl.ds(me, 1)] = x_ref[pl.ds(me * m_per, m_per), :][None]
            for off in range(1, n_dev):
                d = (me + off) % n_dev
                rdma = pltpu.make_async_remote_copy(
                    src_ref=x_block(d),
                    dst_ref=xg_ref.at[me],
                    send_sem=xs_sem.at[d],
                    recv_sem=xr_sem.at[me],
                    device_id=(d,),
                    device_id_type=pl.DeviceIdType.MESH,
                )
                rdma.start()

        @pl.when(j != me)
        def _():
            recv = pltpu.make_async_remote_copy(
                src_ref=x_block(0),
                dst_ref=xg_ref.at[j],
                send_sem=xs_sem.at[j],
                recv_sem=xr_sem.at[j],
                device_id=(me,),
                device_id_type=pl.DeviceIdType.MESH,
            )
            recv.wait_recv()

        xj = xg_ref[pl.ds(j, 1), :, :].reshape(m_per, kc)
        prod = jnp.dot(xj.astype(jnp.bfloat16),
                       w_ref[:, :].astype(jnp.bfloat16),
                       preferred_element_type=jnp.float32)

        @pl.when(j == 0)
        def _():
            out_ref[:, :] = prod

        @pl.when(j != 0)
        def _():
            out_ref[:, :] += prod

        @pl.when(j == n_dev - 1)
        def _():
            for off in range(1, n_dev):
                d = (me + off) % n_dev
                snd = pltpu.make_async_remote_copy(
                    src_ref=x_block(d),
                    dst_ref=xg_ref.at[me],
                    send_sem=xs_sem.at[d],
                    recv_sem=xr_sem.at[me],
                    device_id=(d,),
                    device_id_type=pl.DeviceIdType.MESH,
                )
                snd.wait_send()

            local_amax = jnp.max(jnp.abs(out_ref[:, :]))
            amax_ref[pl.ds(me, 1)] = jnp.full((1, 8, 128), local_amax,
                                              jnp.float32)
            for off in range(1, n_dev):
                d = (me + off) % n_dev
                rdma = pltpu.make_async_remote_copy(
                    src_ref=amax_ref.at[me],
                    dst_ref=amax_ref.at[me],
                    send_sem=as_sem.at[d],
                    recv_sem=ar_sem.at[me],
                    device_id=(d,),
                    device_id_type=pl.DeviceIdType.MESH,
                )
                rdma.start()
            for off in range(1, n_dev):
                s = (me + off) % n_dev
                recv = pltpu.make_async_remote_copy(
                    src_ref=amax_ref.at[me],
                    dst_ref=amax_ref.at[s],
                    send_sem=as_sem.at[s],
                    recv_sem=ar_sem.at[s],
                    device_id=(me,),
                    device_id_type=pl.DeviceIdType.MESH,
                )
                recv.wait_recv()
            for off in range(1, n_dev):
                d = (me + off) % n_dev
                snd = pltpu.make_async_remote_copy(
                    src_ref=amax_ref.at[me],
                    dst_ref=amax_ref.at[me],
                    send_sem=as_sem.at[d],
                    recv_sem=ar_sem.at[me],
                    device_id=(d,),
                    device_id_type=pl.DeviceIdType.MESH,
                )
                snd.wait_send()

            g_amax = jnp.max(amax_ref[:, :, :])
            scale = g_amax / 448.0
            y = out_ref[:, :] / scale
            q = jnp.clip(y, -448.0, 448.0).astype(jnp.float8_e4m3fn)
            out_ref[:, :] = q.astype(jnp.float32) * scale

    return pl.pallas_call(
        body,
        grid=(n_dev,),
        out_shape=jax.ShapeDtypeStruct((m_per, n), jnp.float32),
        in_specs=[
            pl.BlockSpec((k_glob, kc), lambda j: (0, 0),
                         memory_space=pltpu.VMEM),
            pl.BlockSpec((kc, n), lambda j: (j, 0)),
        ],
        out_specs=pl.BlockSpec((m_per, n), lambda j: (0, 0)),
        scratch_shapes=[
            pltpu.VMEM((n_dev, m_per, kc), jnp.float32),
            pltpu.VMEM((n_dev, 8, 128), jnp.float32),
            pltpu.SemaphoreType.DMA((n_dev,)),
            pltpu.SemaphoreType.DMA((n_dev,)),
            pltpu.SemaphoreType.DMA((n_dev,)),
            pltpu.SemaphoreType.DMA((n_dev,)),
        ],
        compiler_params=pltpu.CompilerParams(
            dimension_semantics=("arbitrary",),
        ),
    )(x, w_mat)


# device time: 95400 ns/iter; 1.0881x vs baseline; 1.0881x over previous
import jax
import jax.numpy as jnp
from jax import lax
from jax.experimental import pallas as pl
from jax.experimental.pallas import tpu as pltpu

G = 4
C = 4


def kernel(x, w_mat):
    k_glob, kc = x.shape
    n_dev = k_glob // kc
    m_per = kc
    _, n = w_mat.shape
    kg = k_glob // G
    nc = n // C
    spg = n_dev // G

    def body(x_ref, w_ref, out_ref, xg_ref, amax_ref,
             xs_sem, xr_sem, as_sem, ar_sem):
        g = pl.program_id(0)
        c = pl.program_id(1)
        me = lax.axis_index("i")

        def x_send_block(d):
            return x_ref.at[pl.ds(d * m_per, m_per), :]

        def x_rdma(d, src_slot):
            return pltpu.make_async_remote_copy(
                src_ref=x_send_block(d),
                dst_ref=xg_ref.at[:, pl.ds(src_slot * kc, kc)],
                send_sem=xs_sem.at[d],
                recv_sem=xr_sem.at[src_slot],
                device_id=(d,),
                device_id_type=pl.DeviceIdType.MESH,
            )

        @pl.when(jnp.logical_and(g == 0, c == 0))
        def _():
            xg_ref[:, pl.ds(me * kc, kc)] = x_ref[pl.ds(me * m_per, m_per), :]
            for off in range(1, n_dev):
                d = (me + off) % n_dev
                x_rdma(d, me).start()

        @pl.when(c == 0)
        def _():
            for s0 in range(spg):
                s = g * spg + s0

                @pl.when(s != me)
                def _():
                    x_rdma(me, s).wait_recv()

        a_op = xg_ref[:, pl.ds(g * kg, kg)]
        w_op = w_ref[:, :]
        prod = jnp.dot(a_op, w_op, preferred_element_type=jnp.float32)
        o_sl = (slice(None), pl.ds(c * nc, nc))

        @pl.when(g == 0)
        def _():
            out_ref[o_sl] = prod

        @pl.when(g != 0)
        def _():
            out_ref[o_sl] += prod

        @pl.when(jnp.logical_and(g == G - 1, c == C - 1))
        def _():
            for off in range(1, n_dev):
                d = (me + off) % n_dev
                x_rdma(d, me).wait_send()

            local_amax = jnp.max(jnp.abs(out_ref[:, :]))
            amax_ref[pl.ds(me, 1)] = jnp.full((1, 8, 128), local_amax,
                                              jnp.float32)

            def a_rdma(d, src_slot):
                return pltpu.make_async_remote_copy(
                    src_ref=amax_ref.at[me],
                    dst_ref=amax_ref.at[src_slot],
                    send_sem=as_sem.at[d],
                    recv_sem=ar_sem.at[src_slot],
                    device_id=(d,),
                    device_id_type=pl.DeviceIdType.MESH,
                )

            for off in range(1, n_dev):
                d = (me + off) % n_dev
                a_rdma(d, me).start()
            for off in range(1, n_dev):
                s = (me + off) % n_dev
                a_rdma(me, s).wait_recv()
            for off in range(1, n_dev):
                d = (me + off) % n_dev
                a_rdma(d, me).wait_send()

            g_amax = jnp.max(amax_ref[:, :, :])
            scale = g_amax / 448.0
            y = out_ref[:, :] / scale
            q = jnp.clip(y, -448.0, 448.0).astype(jnp.float8_e4m3fn)
            out_ref[:, :] = q.astype(jnp.float32) * scale

    return pl.pallas_call(
        body,
        grid=(G, C),
        out_shape=jax.ShapeDtypeStruct((m_per, n), jnp.float32),
        in_specs=[
            pl.BlockSpec((k_glob, kc), lambda g, c: (0, 0),
                         memory_space=pltpu.VMEM),
            pl.BlockSpec((kg, nc), lambda g, c: (g, c)),
        ],
        out_specs=pl.BlockSpec((m_per, n), lambda g, c: (0, 0)),
        scratch_shapes=[
            pltpu.VMEM((m_per, k_glob), jnp.float32),
            pltpu.VMEM((n_dev, 8, 128), jnp.float32),
            pltpu.SemaphoreType.DMA((n_dev,)),
            pltpu.SemaphoreType.DMA((n_dev,)),
            pltpu.SemaphoreType.DMA((n_dev,)),
            pltpu.SemaphoreType.DMA((n_dev,)),
        ],
        compiler_params=pltpu.CompilerParams(
            dimension_semantics=("arbitrary", "arbitrary"),
            vmem_limit_bytes=100 * 1024 * 1024,
        ),
    )(x, w_mat)
